# Initial kernel scaffold; baseline (speedup 1.0000x reference)
import jax
import jax.numpy as jnp
from jax.experimental import pallas as pl


def kernel(atomic_numbers, edge_index, edge_length, hopping_param, onsite_param, bond_length_list):
    raise NotImplementedError("write your pallas kernel here")


# speedup vs baseline: 13.1354x; 13.1354x over previous
import functools

import jax
import jax.numpy as jnp
from jax import lax
from jax.experimental import pallas as pl
from jax.experimental.pallas import tpu as pltpu
from jax.experimental.pallas import tpu_sc as plsc

RC = 5.0
W = 1.0
N_TYPES = 2

_NC = 2
_NS = 16
_L = 16


def _pick_chunk(epw: int, cap: int = 12800) -> int:
    best = _L
    c = _L
    while c <= cap:
        if epw % c == 0:
            best = c
        c += _L
    return best


@functools.lru_cache(maxsize=None)
def _make_sc_bond(n_nodes: int, n_edges: int):
    nw = _NC * _NS
    assert n_edges % (nw * _L) == 0
    epw = n_edges // nw
    ch = _pick_chunk(epw)
    n_chunks = epw // ch
    mesh = plsc.VectorSubcoreMesh(core_axis_name="c", subcore_axis_name="s")

    @functools.partial(
        pl.kernel,
        mesh=mesh,
        out_type=jax.ShapeDtypeStruct((n_edges,), jnp.int32),
        compiler_params=pltpu.CompilerParams(needs_layout_passes=False),
        scratch_types=[
            pltpu.VMEM((((n_nodes + 127) // 128) * 128,), jnp.int32),
            pltpu.VMEM((ch,), jnp.int32),
            pltpu.VMEM((ch,), jnp.int32),
            pltpu.VMEM((ch,), jnp.int32),
        ],
    )
    def sc_bond(an_hbm, ei0_hbm, ei1_hbm, out_hbm, an_v, is_v, id_v, bo_v):
        wid = lax.axis_index("s") * _NC + lax.axis_index("c")
        base = wid * epw
        pltpu.sync_copy(an_hbm, an_v.at[pl.ds(0, n_nodes)])
        for c in range(n_chunks):
            off = base + c * ch
            pltpu.sync_copy(ei0_hbm.at[pl.ds(off, ch)], is_v)
            pltpu.sync_copy(ei1_hbm.at[pl.ds(off, ch)], id_v)

            def body(i, carry):
                sl = pl.ds(i * _L, _L)
                zi = plsc.load_gather(an_v, [is_v[sl]])
                zj = plsc.load_gather(an_v, [id_v[sl]])
                bo_v[sl] = zi * N_TYPES + zj
                return carry

            lax.fori_loop(0, ch // _L, body, 0)
            pltpu.sync_copy(bo_v, out_hbm.at[pl.ds(off, ch)])

    return sc_bond


def _tc_edge_body(rij_ref, bond_ref, a1_ref, a2_ref, bl_ref, out_ref):
    rij = rij_ref[...]
    b = bond_ref[...]
    w0 = (b == 0).astype(jnp.float32)
    w1 = (b == 1).astype(jnp.float32)
    w2 = (b == 2).astype(jnp.float32)
    w3 = (b == 3).astype(jnp.float32)
    a1 = (w0 * a1_ref[0:1, :] + w1 * a1_ref[1:2, :]
          + w2 * a1_ref[2:3, :] + w3 * a1_ref[3:4, :])
    a2 = jnp.abs(w0 * a2_ref[0:1, :] + w1 * a2_ref[1:2, :]
                 + w2 * a2_ref[2:3, :] + w3 * a2_ref[3:4, :])
    bl0 = bl_ref[0, 0]
    bl1 = bl_ref[0, 1]
    s = ((b >> 1) + (b & 1)).astype(jnp.float32)
    r0 = bl0 + 0.5 * s * (bl1 - bl0)
    x = jnp.log(r0 / rij)
    fcut = 1.0 / (1.0 + jnp.exp((rij - RC + 5.0 * W) / W))
    out_ref[...] = a1 * jnp.exp((1.0 + a2) * x) * fcut


def _tc_node_body(z_ref, o_ref, out_ref):
    z = z_ref[...].astype(jnp.float32)
    r0 = o_ref[0:1, :]
    r1 = o_ref[1:2, :]
    out_ref[...] = r0 + z * (r1 - r0)


def kernel(atomic_numbers, edge_index, edge_length, hopping_param,
           onsite_param, bond_length_list):
    n_nodes = atomic_numbers.shape[0]
    n_edges = edge_index.shape[1]
    edge_me = hopping_param.shape[1]
    node_me = onsite_param.shape[1]

    an = atomic_numbers.astype(jnp.int32)
    ei = edge_index.astype(jnp.int32)

    bond = _make_sc_bond(n_nodes, n_edges)(an, ei[0], ei[1])

    be = 2000
    grid_e = n_edges // be
    edge_features = pl.pallas_call(
        _tc_edge_body,
        grid=(grid_e,),
        in_specs=[
            pl.BlockSpec((be, 1), lambda i: (i, 0)),
            pl.BlockSpec((be, 1), lambda i: (i, 0)),
            pl.BlockSpec((4, edge_me), lambda i: (0, 0)),
            pl.BlockSpec((4, edge_me), lambda i: (0, 0)),
            pl.BlockSpec((1, 2), lambda i: (0, 0)),
        ],
        out_specs=pl.BlockSpec((be, edge_me), lambda i: (i, 0)),
        out_shape=jax.ShapeDtypeStruct((n_edges, edge_me), jnp.float32),
    )(edge_length.reshape(n_edges, 1),
      bond.reshape(n_edges, 1),
      hopping_param[:, :, 0],
      hopping_param[:, :, 1],
      bond_length_list.reshape(1, 2))

    bn = 2000
    grid_n = n_nodes // bn
    node_features = pl.pallas_call(
        _tc_node_body,
        grid=(grid_n,),
        in_specs=[
            pl.BlockSpec((bn, 1), lambda i: (i, 0)),
            pl.BlockSpec((2, node_me), lambda i: (0, 0)),
        ],
        out_specs=pl.BlockSpec((bn, node_me), lambda i: (i, 0)),
        out_shape=jax.ShapeDtypeStruct((n_nodes, node_me), jnp.float32),
    )(an.reshape(n_nodes, 1), onsite_param[:, :, 0])

    return edge_features, node_features


# speedup vs baseline: 43.7140x; 3.3280x over previous
import functools

import jax
import jax.numpy as jnp
from jax import lax
from jax.experimental import pallas as pl
from jax.experimental.pallas import tpu as pltpu
from jax.experimental.pallas import tpu_sc as plsc

RC = 5.0
W = 1.0
N_TYPES = 2

_NC = 2
_NS = 16
_L = 16


def _pick_chunk(epw: int, cap: int = 12800) -> int:
    best = _L
    c = _L
    while c <= cap:
        if epw % c == 0:
            best = c
        c += _L
    return best


@functools.lru_cache(maxsize=None)
def _make_sc_bond(n_nodes: int, n_edges: int):
    nw = _NC * _NS
    assert n_edges % (nw * _L) == 0
    epw = n_edges // nw
    ch = _pick_chunk(epw)
    n_chunks = epw // ch
    mesh = plsc.VectorSubcoreMesh(core_axis_name="c", subcore_axis_name="s")

    @functools.partial(
        pl.kernel,
        mesh=mesh,
        out_type=jax.ShapeDtypeStruct((n_edges,), jnp.int32),
        compiler_params=pltpu.CompilerParams(needs_layout_passes=False),
        scratch_types=[
            pltpu.VMEM((((n_nodes + 127) // 128) * 128,), jnp.int32),
            pltpu.VMEM((ch,), jnp.int32),
            pltpu.VMEM((ch,), jnp.int32),
            pltpu.VMEM((ch,), jnp.int32),
        ],
    )
    def sc_bond(an_hbm, ei0_hbm, ei1_hbm, out_hbm, an_v, is_v, id_v, bo_v):
        wid = lax.axis_index("s") * _NC + lax.axis_index("c")
        base = wid * epw
        pltpu.sync_copy(an_hbm, an_v.at[pl.ds(0, n_nodes)])
        for c in range(n_chunks):
            off = base + c * ch
            pltpu.sync_copy(ei0_hbm.at[pl.ds(off, ch)], is_v)
            pltpu.sync_copy(ei1_hbm.at[pl.ds(off, ch)], id_v)

            def body(i, carry):
                sl = pl.ds(i * _L, _L)
                zi = plsc.load_gather(an_v, [is_v[sl]])
                zj = plsc.load_gather(an_v, [id_v[sl]])
                bo_v[sl] = zi * N_TYPES + zj
                return carry

            lax.fori_loop(0, ch // _L, body, 0)
            pltpu.sync_copy(bo_v, out_hbm.at[pl.ds(off, ch)])

    return sc_bond


def _tc_edge_body(rij_ref, bond_ref, a1_ref, a2_ref, bl_ref, out_ref):
    be = rij_ref.shape[0]
    rij = rij_ref[...].reshape(1, be)
    b = bond_ref[...].reshape(1, be)
    w0 = (b == 0).astype(jnp.float32)
    w1 = (b == 1).astype(jnp.float32)
    w2 = (b == 2).astype(jnp.float32)
    w3 = (b == 3).astype(jnp.float32)
    a1 = (a1_ref[:, 0:1] * w0 + a1_ref[:, 1:2] * w1
          + a1_ref[:, 2:3] * w2 + a1_ref[:, 3:4] * w3)
    a2 = jnp.abs(a2_ref[:, 0:1] * w0 + a2_ref[:, 1:2] * w1
                 + a2_ref[:, 2:3] * w2 + a2_ref[:, 3:4] * w3)
    bl0 = bl_ref[0, 0]
    bl1 = bl_ref[0, 1]
    s = ((b >> 1) + (b & 1)).astype(jnp.float32)
    r0 = bl0 + 0.5 * s * (bl1 - bl0)
    x = jnp.log(r0 / rij)
    fcut = 1.0 / (1.0 + jnp.exp((rij - RC + 5.0 * W) / W))
    out_t = a1 * jnp.exp(x + a2 * x) * fcut
    out_ref[...] = lax.transpose(out_t, (1, 0))[:, :out_ref.shape[1]]


def _tc_node_body(z_ref, o_ref, out_ref):
    z = z_ref[...].astype(jnp.float32)
    r0 = o_ref[0:1, :]
    r1 = o_ref[1:2, :]
    out_ref[...] = r0 + z * (r1 - r0)


def kernel(atomic_numbers, edge_index, edge_length, hopping_param,
           onsite_param, bond_length_list):
    n_nodes = atomic_numbers.shape[0]
    n_edges = edge_index.shape[1]
    edge_me = hopping_param.shape[1]
    node_me = onsite_param.shape[1]

    an = atomic_numbers.astype(jnp.int32)
    ei = edge_index.astype(jnp.int32)

    bond = _make_sc_bond(n_nodes, n_edges)(an, ei[0], ei[1])

    a1t = jnp.zeros((16, 4), jnp.float32).at[:edge_me, :].set(
        hopping_param[:, :, 0].T)
    a2t = jnp.zeros((16, 4), jnp.float32).at[:edge_me, :].set(
        hopping_param[:, :, 1].T)
    be = 4096
    grid_e = pl.cdiv(n_edges, be)
    edge_features = pl.pallas_call(
        _tc_edge_body,
        grid=(grid_e,),
        in_specs=[
            pl.BlockSpec((be,), lambda i: (i,)),
            pl.BlockSpec((be,), lambda i: (i,)),
            pl.BlockSpec((16, 4), lambda i: (0, 0)),
            pl.BlockSpec((16, 4), lambda i: (0, 0)),
            pl.BlockSpec((1, 2), lambda i: (0, 0)),
        ],
        out_specs=pl.BlockSpec((be, edge_me), lambda i: (i, 0)),
        out_shape=jax.ShapeDtypeStruct((n_edges, edge_me), jnp.float32),
    )(edge_length, bond, a1t, a2t, bond_length_list.reshape(1, 2))

    bn = 2000
    grid_n = n_nodes // bn
    node_features = pl.pallas_call(
        _tc_node_body,
        grid=(grid_n,),
        in_specs=[
            pl.BlockSpec((bn, 1), lambda i: (i, 0)),
            pl.BlockSpec((2, node_me), lambda i: (0, 0)),
        ],
        out_specs=pl.BlockSpec((bn, node_me), lambda i: (i, 0)),
        out_shape=jax.ShapeDtypeStruct((n_nodes, node_me), jnp.float32),
    )(an.reshape(n_nodes, 1), onsite_param[:, :, 0])

    return edge_features, node_features


# speedup vs baseline: 151.3820x; 3.4630x over previous
import functools

import jax
import jax.numpy as jnp
from jax import lax
from jax.experimental import pallas as pl
from jax.experimental.pallas import tpu as pltpu
from jax.experimental.pallas import tpu_sc as plsc

RC = 5.0
W = 1.0
N_TYPES = 2

_NC = 2
_NS = 16
_L = 16


def _pick_chunk(epw: int, cap: int = 12800) -> int:
    best = _L
    c = _L
    while c <= cap:
        if epw % c == 0:
            best = c
        c += _L
    return best


@functools.lru_cache(maxsize=None)
def _make_sc_bond(n_nodes: int, n_edges: int):
    nw = _NC * _NS
    assert n_edges % (nw * _L) == 0
    epw = n_edges // nw
    ch = _pick_chunk(epw)
    n_chunks = epw // ch
    mesh = plsc.VectorSubcoreMesh(core_axis_name="c", subcore_axis_name="s")

    @functools.partial(
        pl.kernel,
        mesh=mesh,
        out_type=jax.ShapeDtypeStruct((n_edges,), jnp.int32),
        compiler_params=pltpu.CompilerParams(needs_layout_passes=False),
        scratch_types=[
            pltpu.VMEM((((n_nodes + 127) // 128) * 128,), jnp.int32),
            pltpu.VMEM((ch,), jnp.int32),
            pltpu.VMEM((ch,), jnp.int32),
            pltpu.VMEM((ch,), jnp.int32),
        ],
    )
    def sc_bond(an_hbm, ei_hbm, out_hbm, an_v, is_v, id_v, bo_v):
        wid = lax.axis_index("s") * _NC + lax.axis_index("c")
        base = wid * epw
        pltpu.sync_copy(an_hbm, an_v.at[pl.ds(0, n_nodes)])
        for c in range(n_chunks):
            off = base + c * ch
            pltpu.sync_copy(ei_hbm.at[pl.ds(off, ch)], is_v)
            pltpu.sync_copy(ei_hbm.at[pl.ds(n_edges + off, ch)], id_v)

            def body(i, carry):
                sl = pl.ds(i * _L, _L)
                zi = plsc.load_gather(an_v, [is_v[sl]])
                zj = plsc.load_gather(an_v, [id_v[sl]])
                bo_v[sl] = zi * N_TYPES + zj
                return carry

            lax.fori_loop(0, ch // _L, body, 0)
            pltpu.sync_copy(bo_v, out_hbm.at[pl.ds(off, ch)])

    return sc_bond


def _tc_edge_body(rij_ref, bond_ref, a1_ref, a2_ref, bl_ref, out_ref):
    be = rij_ref.shape[0]
    rij = rij_ref[...].reshape(1, be)
    b = bond_ref[...].reshape(1, be)
    w0 = (b == 0).astype(jnp.float32)
    w1 = (b == 1).astype(jnp.float32)
    w2 = (b == 2).astype(jnp.float32)
    w3 = (b == 3).astype(jnp.float32)
    a1 = (a1_ref[:, 0:1] * w0 + a1_ref[:, 1:2] * w1
          + a1_ref[:, 2:3] * w2 + a1_ref[:, 3:4] * w3)
    a2 = jnp.abs(a2_ref[:, 0:1] * w0 + a2_ref[:, 1:2] * w1
                 + a2_ref[:, 2:3] * w2 + a2_ref[:, 3:4] * w3)
    bl0 = bl_ref[0, 0]
    bl1 = bl_ref[0, 1]
    s = ((b >> 1) + (b & 1)).astype(jnp.float32)
    r0 = bl0 + 0.5 * s * (bl1 - bl0)
    x = jnp.log(r0 / rij)
    fcut = 1.0 / (1.0 + jnp.exp((rij - RC + 5.0 * W) / W))
    out_t = a1 * jnp.exp(x + a2 * x) * fcut
    out_ref[...] = out_t[:out_ref.shape[0], :]


def _tc_node_body(z_ref, o_ref, out_ref):
    bn = z_ref.shape[0]
    z = z_ref[...].reshape(1, bn).astype(jnp.float32)
    c0 = o_ref[:, 0:1]
    c1 = o_ref[:, 1:2]
    nf_t = c0 + z * (c1 - c0)
    out_ref[...] = nf_t[:out_ref.shape[0], :]


def kernel(atomic_numbers, edge_index, edge_length, hopping_param,
           onsite_param, bond_length_list):
    n_nodes = atomic_numbers.shape[0]
    n_edges = edge_index.shape[1]
    edge_me = hopping_param.shape[1]
    node_me = onsite_param.shape[1]

    an = atomic_numbers.astype(jnp.int32)
    ei = edge_index.astype(jnp.int32)

    bond = _make_sc_bond(n_nodes, n_edges)(an, ei.reshape(-1))

    a1t = jnp.zeros((16, 4), jnp.float32).at[:edge_me, :].set(
        hopping_param[:, :, 0].T)
    a2t = jnp.zeros((16, 4), jnp.float32).at[:edge_me, :].set(
        hopping_param[:, :, 1].T)
    ot = jnp.zeros((8, 2), jnp.float32).at[:node_me, :].set(
        onsite_param[:, :, 0].T)

    be = 32768
    grid_e = pl.cdiv(n_edges, be)
    ef_t = pl.pallas_call(
        _tc_edge_body,
        grid=(grid_e,),
        in_specs=[
            pl.BlockSpec((be,), lambda i: (i,)),
            pl.BlockSpec((be,), lambda i: (i,)),
            pl.BlockSpec((16, 4), lambda i: (0, 0)),
            pl.BlockSpec((16, 4), lambda i: (0, 0)),
            pl.BlockSpec((1, 2), lambda i: (0, 0)),
        ],
        out_specs=pl.BlockSpec((edge_me, be), lambda i: (0, i)),
        out_shape=jax.ShapeDtypeStruct((edge_me, n_edges), jnp.float32),
    )(edge_length, bond, a1t, a2t, bond_length_list.reshape(1, 2))

    bn = 2048
    grid_n = pl.cdiv(n_nodes, bn)
    nf_t = pl.pallas_call(
        _tc_node_body,
        grid=(grid_n,),
        in_specs=[
            pl.BlockSpec((bn,), lambda i: (i,)),
            pl.BlockSpec((8, 2), lambda i: (0, 0)),
        ],
        out_specs=pl.BlockSpec((node_me, bn), lambda i: (0, i)),
        out_shape=jax.ShapeDtypeStruct((node_me, n_nodes), jnp.float32),
    )(an, ot)

    return ef_t.T, nf_t.T


# speedup vs baseline: 295.4887x; 1.9519x over previous
import functools

import jax
import jax.numpy as jnp
from jax import lax
from jax.experimental import pallas as pl
from jax.experimental.pallas import tpu as pltpu
from jax.experimental.pallas import tpu_sc as plsc

RC = 5.0
W = 1.0
N_TYPES = 2

_NC = 2
_NS = 16
_L = 16


def _pick_chunk(epw: int, cap: int = 12800) -> int:
    best = _L
    c = _L
    while c <= cap:
        if epw % c == 0:
            best = c
        c += _L
    return best


@functools.lru_cache(maxsize=None)
def _make_sc_bond(n_nodes: int, n_edges: int):
    nw = _NC * _NS
    assert n_edges % (nw * _L) == 0
    epw = n_edges // nw
    ch = _pick_chunk(epw)
    n_chunks = epw // ch
    mesh = plsc.VectorSubcoreMesh(core_axis_name="c", subcore_axis_name="s")

    @functools.partial(
        pl.kernel,
        mesh=mesh,
        out_type=jax.ShapeDtypeStruct((n_edges,), jnp.int32),
        compiler_params=pltpu.CompilerParams(needs_layout_passes=False),
        scratch_types=[
            pltpu.VMEM((((n_nodes + 127) // 128) * 128,), jnp.int32),
            pltpu.VMEM((ch,), jnp.int32),
            pltpu.VMEM((ch,), jnp.int32),
            pltpu.VMEM((ch,), jnp.int32),
        ],
    )
    def sc_bond(an_hbm, ei_hbm, out_hbm, an_v, is_v, id_v, bo_v):
        wid = lax.axis_index("s") * _NC + lax.axis_index("c")
        base = wid * epw
        pltpu.sync_copy(an_hbm, an_v.at[pl.ds(0, n_nodes)])
        for c in range(n_chunks):
            off = base + c * ch
            pltpu.sync_copy(ei_hbm.at[pl.ds(off, ch)], is_v)
            pltpu.sync_copy(ei_hbm.at[pl.ds(n_edges + off, ch)], id_v)

            def body(i, carry):
                sl = pl.ds(i * _L, _L)
                zi = plsc.load_gather(an_v, [is_v[sl]])
                zj = plsc.load_gather(an_v, [id_v[sl]])
                bo_v[sl] = zi * N_TYPES + zj
                return carry

            lax.fori_loop(0, ch // _L, body, 0)
            pltpu.sync_copy(bo_v, out_hbm.at[pl.ds(off, ch)])

    return sc_bond


def _tc_edge_body(rij_ref, bond_ref, a1_ref, a2_ref, bl_ref, out_ref):
    be = rij_ref.shape[0]
    rij = rij_ref[...].reshape(1, be)
    b = bond_ref[...].reshape(1, be)
    rows = lax.broadcasted_iota(jnp.int32, (8, be), 0)
    w_oh = (rows == b).astype(jnp.float32)
    fcut = 1.0 / (1.0 + jnp.exp((rij - RC + 5.0 * W) / W))
    a1f = jnp.dot(a1_ref[...], w_oh * fcut,
                  preferred_element_type=jnp.float32)
    a2p = jnp.dot(jnp.abs(a2_ref[...]) + 1.0, w_oh,
                  preferred_element_type=jnp.float32)
    bl0 = bl_ref[0, 0]
    bl1 = bl_ref[0, 1]
    s = ((b >> 1) + (b & 1)).astype(jnp.float32)
    r0 = bl0 + 0.5 * s * (bl1 - bl0)
    x = jnp.log(r0 / rij)
    out_t = a1f * jnp.exp(a2p * x)
    out_ref[...] = out_t[:out_ref.shape[0], :]


def _tc_node_body(z_ref, o_ref, out_ref):
    bn = z_ref.shape[0]
    z = z_ref[...].reshape(1, bn).astype(jnp.float32)
    c0 = o_ref[:, 0:1]
    c1 = o_ref[:, 1:2]
    nf_t = c0 + z * (c1 - c0)
    out_ref[...] = nf_t[:out_ref.shape[0], :]


def kernel(atomic_numbers, edge_index, edge_length, hopping_param,
           onsite_param, bond_length_list):
    n_nodes = atomic_numbers.shape[0]
    n_edges = edge_index.shape[1]
    edge_me = hopping_param.shape[1]
    node_me = onsite_param.shape[1]

    an = atomic_numbers.astype(jnp.int32)
    ei = edge_index.astype(jnp.int32)

    bond = _make_sc_bond(n_nodes, n_edges)(an, ei.reshape(-1))

    a1t = jnp.zeros((16, 8), jnp.float32).at[:edge_me, :4].set(
        hopping_param[:, :, 0].T)
    a2t = jnp.zeros((16, 8), jnp.float32).at[:edge_me, :4].set(
        hopping_param[:, :, 1].T)
    ot = jnp.zeros((8, 2), jnp.float32).at[:node_me, :].set(
        onsite_param[:, :, 0].T)

    be = 32768
    grid_e = pl.cdiv(n_edges, be)
    ef_t = pl.pallas_call(
        _tc_edge_body,
        grid=(grid_e,),
        in_specs=[
            pl.BlockSpec((be,), lambda i: (i,)),
            pl.BlockSpec((be,), lambda i: (i,)),
            pl.BlockSpec((16, 8), lambda i: (0, 0)),
            pl.BlockSpec((16, 8), lambda i: (0, 0)),
            pl.BlockSpec((1, 2), lambda i: (0, 0)),
        ],
        out_specs=pl.BlockSpec((edge_me, be), lambda i: (0, i)),
        out_shape=jax.ShapeDtypeStruct((edge_me, n_edges), jnp.float32),
    )(edge_length, bond, a1t, a2t, bond_length_list.reshape(1, 2))

    bn = 2048
    grid_n = pl.cdiv(n_nodes, bn)
    nf_t = pl.pallas_call(
        _tc_node_body,
        grid=(grid_n,),
        in_specs=[
            pl.BlockSpec((bn,), lambda i: (i,)),
            pl.BlockSpec((8, 2), lambda i: (0, 0)),
        ],
        out_specs=pl.BlockSpec((node_me, bn), lambda i: (0, i)),
        out_shape=jax.ShapeDtypeStruct((node_me, n_nodes), jnp.float32),
    )(an, ot)

    return ef_t.T, nf_t.T


# speedup vs baseline: 384.7336x; 1.3020x over previous
import functools

import jax
import jax.numpy as jnp
from jax import lax
from jax.experimental import pallas as pl
from jax.experimental.pallas import tpu as pltpu
from jax.experimental.pallas import tpu_sc as plsc

RC = 5.0
W = 1.0
N_TYPES = 2

_NC = 2
_NS = 16
_L = 16


@functools.lru_cache(maxsize=None)
def _make_sc_bond(n_nodes: int, n_edges: int):
    nw = _NC * _NS
    ch = 6400
    assert n_edges % ch == 0
    n_chunks = n_edges // ch
    max_k = -(-n_chunks // nw)
    mesh = plsc.VectorSubcoreMesh(core_axis_name="c", subcore_axis_name="s")

    @functools.partial(
        pl.kernel,
        mesh=mesh,
        out_type=jax.ShapeDtypeStruct((n_edges,), jnp.int32),
        compiler_params=pltpu.CompilerParams(needs_layout_passes=False),
        scratch_types=[
            pltpu.VMEM((((n_nodes + 127) // 128) * 128,), jnp.int32),
            pltpu.VMEM((2, 2, ch), jnp.int32),
            pltpu.VMEM((2, ch), jnp.int32),
            pltpu.SemaphoreType.DMA,
            pltpu.SemaphoreType.DMA,
            pltpu.SemaphoreType.DMA,
            pltpu.SemaphoreType.DMA,
        ],
    )
    def sc_bond(an_hbm, ei_hbm, out_hbm, an_v, in_v, bo_v, si0, si1, so0, so1):
        wid = lax.axis_index("s") * _NC + lax.axis_index("c")
        sems_in = (si0, si1)
        sems_out = (so0, so1)
        pltpu.sync_copy(an_hbm, an_v.at[pl.ds(0, n_nodes)])

        def start_in(k):
            c = wid + nw * k

            @pl.when(c < n_chunks)
            def _():
                pltpu.async_copy(ei_hbm.at[:, pl.ds(c * ch, ch)],
                                 in_v.at[k % 2], sems_in[k % 2])

        start_in(0)
        for k in range(max_k):
            buf = k % 2
            c = wid + nw * k
            if k + 1 < max_k:
                start_in(k + 1)

            @pl.when(c < n_chunks)
            def _():
                pltpu.make_async_copy(ei_hbm.at[:, pl.ds(c * ch, ch)],
                                      in_v.at[buf], sems_in[buf]).wait()
                if k >= 2:
                    pltpu.make_async_copy(
                        bo_v.at[buf],
                        out_hbm.at[pl.ds((c - 2 * nw) * ch, ch)],
                        sems_out[buf]).wait()

                def body(i, carry):
                    sl = pl.ds(i * _L, _L)
                    zi = plsc.load_gather(an_v, [in_v[buf, 0, sl]])
                    zj = plsc.load_gather(an_v, [in_v[buf, 1, sl]])
                    bo_v[buf, sl] = zi * N_TYPES + zj
                    return carry

                lax.fori_loop(0, ch // _L, body, 0)
                pltpu.async_copy(bo_v.at[buf],
                                 out_hbm.at[pl.ds(c * ch, ch)], sems_out[buf])

        for k in range(max(0, max_k - 3), max_k):
            buf = k % 2
            c = wid + nw * k

            @pl.when((c < n_chunks) & (c + 2 * nw >= n_chunks))
            def _():
                pltpu.make_async_copy(bo_v.at[buf],
                                      out_hbm.at[pl.ds(c * ch, ch)],
                                      sems_out[buf]).wait()

    return sc_bond


def _tc_edge_body(rij_ref, bond_ref, a1_ref, a2_ref, bl_ref, out_ref):
    be = rij_ref.shape[0]
    rij = rij_ref[...].reshape(1, be)
    b = bond_ref[...].reshape(1, be)
    rows = lax.broadcasted_iota(jnp.int32, (8, be), 0)
    w_oh = (rows == b).astype(jnp.float32)
    fcut = 1.0 / (1.0 + jnp.exp((rij - RC + 5.0 * W) / W))
    a1f = jnp.dot(a1_ref[...], w_oh * fcut,
                  preferred_element_type=jnp.float32)
    a2p = jnp.dot(jnp.abs(a2_ref[...]) + 1.0, w_oh,
                  preferred_element_type=jnp.float32)
    bl0 = bl_ref[0, 0]
    bl1 = bl_ref[0, 1]
    s = ((b >> 1) + (b & 1)).astype(jnp.float32)
    r0 = bl0 + 0.5 * s * (bl1 - bl0)
    x = jnp.log(r0 / rij)
    out_t = a1f * jnp.exp(a2p * x)
    out_ref[...] = out_t[:out_ref.shape[0], :]


def _tc_node_body(z_ref, o_ref, out_ref):
    bn = z_ref.shape[0]
    z = z_ref[...].reshape(1, bn).astype(jnp.float32)
    c0 = o_ref[:, 0:1]
    c1 = o_ref[:, 1:2]
    nf_t = c0 + z * (c1 - c0)
    out_ref[...] = nf_t[:out_ref.shape[0], :]


def kernel(atomic_numbers, edge_index, edge_length, hopping_param,
           onsite_param, bond_length_list):
    n_nodes = atomic_numbers.shape[0]
    n_edges = edge_index.shape[1]
    edge_me = hopping_param.shape[1]
    node_me = onsite_param.shape[1]

    an = atomic_numbers.astype(jnp.int32)
    ei = edge_index.astype(jnp.int32)

    bond = _make_sc_bond(n_nodes, n_edges)(an, ei)

    a1t = jnp.zeros((16, 8), jnp.float32).at[:edge_me, :4].set(
        hopping_param[:, :, 0].T)
    a2t = jnp.zeros((16, 8), jnp.float32).at[:edge_me, :4].set(
        hopping_param[:, :, 1].T)
    ot = jnp.zeros((8, 2), jnp.float32).at[:node_me, :].set(
        onsite_param[:, :, 0].T)

    be = 32768
    grid_e = pl.cdiv(n_edges, be)
    ef_t = pl.pallas_call(
        _tc_edge_body,
        grid=(grid_e,),
        in_specs=[
            pl.BlockSpec((be,), lambda i: (i,)),
            pl.BlockSpec((be,), lambda i: (i,)),
            pl.BlockSpec((16, 8), lambda i: (0, 0)),
            pl.BlockSpec((16, 8), lambda i: (0, 0)),
            pl.BlockSpec((1, 2), lambda i: (0, 0)),
        ],
        out_specs=pl.BlockSpec((edge_me, be), lambda i: (0, i)),
        out_shape=jax.ShapeDtypeStruct((edge_me, n_edges), jnp.float32),
    )(edge_length, bond, a1t, a2t, bond_length_list.reshape(1, 2))

    nf_t = pl.pallas_call(
        _tc_node_body,
        in_specs=[
            pl.BlockSpec((n_nodes,), lambda: (0,)),
            pl.BlockSpec((8, 2), lambda: (0, 0)),
        ],
        out_specs=pl.BlockSpec((node_me, n_nodes), lambda: (0, 0)),
        out_shape=jax.ShapeDtypeStruct((node_me, n_nodes), jnp.float32),
    )(an, ot)

    return ef_t.T, nf_t.T


# speedup vs baseline: 419.5487x; 1.0905x over previous
import functools

import jax
import jax.numpy as jnp
from jax import lax
from jax.experimental import pallas as pl
from jax.experimental.pallas import tpu as pltpu
from jax.experimental.pallas import tpu_sc as plsc

RC = 5.0
W = 1.0
N_TYPES = 2

_NC = 2
_NS = 16
_L = 16


@functools.lru_cache(maxsize=None)
def _make_sc_bond(n_nodes: int, n_edges: int):
    nw = _NC * _NS
    ch = 6400
    assert n_edges % ch == 0
    n_chunks = n_edges // ch
    max_k = -(-n_chunks // nw)
    mesh = plsc.VectorSubcoreMesh(core_axis_name="c", subcore_axis_name="s")

    @functools.partial(
        pl.kernel,
        mesh=mesh,
        out_type=jax.ShapeDtypeStruct((n_edges,), jnp.int32),
        compiler_params=pltpu.CompilerParams(needs_layout_passes=False),
        scratch_types=[
            pltpu.VMEM((((n_nodes + 127) // 128) * 128,), jnp.int32),
            pltpu.VMEM((2, 2, ch), jnp.int32),
            pltpu.VMEM((2, ch), jnp.int32),
            pltpu.SemaphoreType.DMA,
            pltpu.SemaphoreType.DMA,
            pltpu.SemaphoreType.DMA,
            pltpu.SemaphoreType.DMA,
        ],
    )
    def sc_bond(an_hbm, ei_hbm, out_hbm, an_v, in_v, bo_v, si0, si1, so0, so1):
        wid = lax.axis_index("s") * _NC + lax.axis_index("c")
        sems_in = (si0, si1)
        sems_out = (so0, so1)
        pltpu.sync_copy(an_hbm, an_v.at[pl.ds(0, n_nodes)])

        def start_in(k):
            c = wid + nw * k

            @pl.when(c < n_chunks)
            def _():
                pltpu.async_copy(ei_hbm.at[:, pl.ds(c * ch, ch)],
                                 in_v.at[k % 2], sems_in[k % 2])

        start_in(0)
        for k in range(max_k):
            buf = k % 2
            c = wid + nw * k
            if k + 1 < max_k:
                start_in(k + 1)

            @pl.when(c < n_chunks)
            def _():
                pltpu.make_async_copy(ei_hbm.at[:, pl.ds(c * ch, ch)],
                                      in_v.at[buf], sems_in[buf]).wait()
                if k >= 2:
                    pltpu.make_async_copy(
                        bo_v.at[buf],
                        out_hbm.at[pl.ds((c - 2 * nw) * ch, ch)],
                        sems_out[buf]).wait()

                @plsc.parallel_loop(0, ch, step=_L, unroll=8)
                def body(i):
                    sl = pl.ds(i, _L)
                    zi = plsc.load_gather(an_v, [in_v[buf, 0, sl]])
                    zj = plsc.load_gather(an_v, [in_v[buf, 1, sl]])
                    bo_v[buf, sl] = zi * N_TYPES + zj
                pltpu.async_copy(bo_v.at[buf],
                                 out_hbm.at[pl.ds(c * ch, ch)], sems_out[buf])

        for k in range(max(0, max_k - 3), max_k):
            buf = k % 2
            c = wid + nw * k

            @pl.when((c < n_chunks) & (c + 2 * nw >= n_chunks))
            def _():
                pltpu.make_async_copy(bo_v.at[buf],
                                      out_hbm.at[pl.ds(c * ch, ch)],
                                      sems_out[buf]).wait()

    return sc_bond


def _tc_edge_body(rij_ref, bond_ref, a1_ref, a2_ref, bl_ref, out_ref):
    be = rij_ref.shape[0]
    rij = rij_ref[...].reshape(1, be)
    b = bond_ref[...].reshape(1, be)
    rows = lax.broadcasted_iota(jnp.int32, (8, be), 0)
    w_oh = (rows == b).astype(jnp.float32)
    fcut = 1.0 / (1.0 + jnp.exp((rij - RC + 5.0 * W) / W))
    a1f = jnp.dot(a1_ref[...], w_oh * fcut,
                  preferred_element_type=jnp.float32)
    a2p = jnp.dot(jnp.abs(a2_ref[...]) + 1.0, w_oh,
                  preferred_element_type=jnp.float32)
    bl0 = bl_ref[0, 0]
    bl1 = bl_ref[0, 1]
    s = ((b >> 1) + (b & 1)).astype(jnp.float32)
    r0 = bl0 + 0.5 * s * (bl1 - bl0)
    x = jnp.log(r0 / rij)
    out_t = a1f * jnp.exp(a2p * x)
    out_ref[...] = out_t[:out_ref.shape[0], :]


def _tc_node_body(z_ref, o_ref, out_ref):
    bn = z_ref.shape[0]
    z = z_ref[...].reshape(1, bn).astype(jnp.float32)
    c0 = o_ref[:, 0:1]
    c1 = o_ref[:, 1:2]
    nf_t = c0 + z * (c1 - c0)
    out_ref[...] = nf_t[:out_ref.shape[0], :]


def kernel(atomic_numbers, edge_index, edge_length, hopping_param,
           onsite_param, bond_length_list):
    n_nodes = atomic_numbers.shape[0]
    n_edges = edge_index.shape[1]
    edge_me = hopping_param.shape[1]
    node_me = onsite_param.shape[1]

    an = atomic_numbers.astype(jnp.int32)
    ei = edge_index.astype(jnp.int32)

    bond = _make_sc_bond(n_nodes, n_edges)(an, ei)

    a1t = jnp.zeros((16, 8), jnp.float32).at[:edge_me, :4].set(
        hopping_param[:, :, 0].T)
    a2t = jnp.zeros((16, 8), jnp.float32).at[:edge_me, :4].set(
        hopping_param[:, :, 1].T)
    ot = jnp.zeros((8, 2), jnp.float32).at[:node_me, :].set(
        onsite_param[:, :, 0].T)

    be = 32768
    grid_e = pl.cdiv(n_edges, be)
    ef_t = pl.pallas_call(
        _tc_edge_body,
        grid=(grid_e,),
        in_specs=[
            pl.BlockSpec((be,), lambda i: (i,)),
            pl.BlockSpec((be,), lambda i: (i,)),
            pl.BlockSpec((16, 8), lambda i: (0, 0)),
            pl.BlockSpec((16, 8), lambda i: (0, 0)),
            pl.BlockSpec((1, 2), lambda i: (0, 0)),
        ],
        out_specs=pl.BlockSpec((edge_me, be), lambda i: (0, i)),
        out_shape=jax.ShapeDtypeStruct((edge_me, n_edges), jnp.float32),
    )(edge_length, bond, a1t, a2t, bond_length_list.reshape(1, 2))

    nf_t = pl.pallas_call(
        _tc_node_body,
        in_specs=[
            pl.BlockSpec((n_nodes,), lambda: (0,)),
            pl.BlockSpec((8, 2), lambda: (0, 0)),
        ],
        out_specs=pl.BlockSpec((node_me, n_nodes), lambda: (0, 0)),
        out_shape=jax.ShapeDtypeStruct((node_me, n_nodes), jnp.float32),
    )(an, ot)

    return ef_t.T, nf_t.T


# speedup vs baseline: 474.4568x; 1.1309x over previous
import functools

import jax
import jax.numpy as jnp
from jax import lax
from jax.experimental import pallas as pl
from jax.experimental.pallas import tpu as pltpu
from jax.experimental.pallas import tpu_sc as plsc

RC = 5.0
W = 1.0
N_TYPES = 2

_NC = 2
_NS = 16
_L = 16


@functools.lru_cache(maxsize=None)
def _make_sc_bond(n_nodes: int, n_edges: int):
    nw = _NC * _NS
    ch = 6400
    assert n_edges % ch == 0
    n_chunks = n_edges // ch
    max_k = -(-n_chunks // nw)
    mesh = plsc.VectorSubcoreMesh(core_axis_name="c", subcore_axis_name="s")

    @functools.partial(
        pl.kernel,
        mesh=mesh,
        out_type=jax.ShapeDtypeStruct((n_edges,), jnp.int32),
        compiler_params=pltpu.CompilerParams(needs_layout_passes=False),
        scratch_types=[
            pltpu.VMEM((((n_nodes + 127) // 128) * 128,), jnp.int32),
            pltpu.VMEM((2, 2, ch), jnp.int32),
            pltpu.VMEM((2, ch), jnp.int32),
            pltpu.SemaphoreType.DMA,
            pltpu.SemaphoreType.DMA,
            pltpu.SemaphoreType.DMA,
            pltpu.SemaphoreType.DMA,
        ],
    )
    def sc_bond(an_hbm, ei_hbm, out_hbm, an_v, in_v, bo_v, si0, si1, so0, so1):
        wid = lax.axis_index("s") * _NC + lax.axis_index("c")
        sems_in = (si0, si1)
        sems_out = (so0, so1)
        pltpu.sync_copy(an_hbm, an_v.at[pl.ds(0, n_nodes)])

        def start_in(k):
            c = wid + nw * k

            @pl.when(c < n_chunks)
            def _():
                pltpu.async_copy(ei_hbm.at[:, pl.ds(c * ch, ch)],
                                 in_v.at[k % 2], sems_in[k % 2])

        start_in(0)
        for k in range(max_k):
            buf = k % 2
            c = wid + nw * k
            if k + 1 < max_k:
                start_in(k + 1)

            @pl.when(c < n_chunks)
            def _():
                pltpu.make_async_copy(ei_hbm.at[:, pl.ds(c * ch, ch)],
                                      in_v.at[buf], sems_in[buf]).wait()
                if k >= 2:
                    pltpu.make_async_copy(
                        bo_v.at[buf],
                        out_hbm.at[pl.ds((c - 2 * nw) * ch, ch)],
                        sems_out[buf]).wait()

                @plsc.parallel_loop(0, ch, step=_L, unroll=8)
                def body(i):
                    sl = pl.ds(i, _L)
                    zi = plsc.load_gather(an_v, [in_v[buf, 0, sl]])
                    zj = plsc.load_gather(an_v, [in_v[buf, 1, sl]])
                    bo_v[buf, sl] = zi * N_TYPES + zj
                pltpu.async_copy(bo_v.at[buf],
                                 out_hbm.at[pl.ds(c * ch, ch)], sems_out[buf])

        for k in range(max(0, max_k - 3), max_k):
            buf = k % 2
            c = wid + nw * k

            @pl.when((c < n_chunks) & (c + 2 * nw >= n_chunks))
            def _():
                pltpu.make_async_copy(bo_v.at[buf],
                                      out_hbm.at[pl.ds(c * ch, ch)],
                                      sems_out[buf]).wait()

    return sc_bond


def _tc_edge_body(rij_ref, bond_ref, a1_ref, a2_ref, bl_ref, out_ref):
    be = rij_ref.shape[0]
    rij = rij_ref[...].reshape(1, be)
    b = bond_ref[...].reshape(1, be)
    rows = lax.broadcasted_iota(jnp.int32, (8, be), 0)
    w_oh = (rows == b).astype(jnp.float32)
    fcut = 1.0 / (1.0 + jnp.exp((rij - RC + 5.0 * W) / W))
    a1f = jnp.dot(a1_ref[...], w_oh * fcut,
                  preferred_element_type=jnp.float32)
    a2p = jnp.dot(jnp.abs(a2_ref[...]) + 1.0, w_oh,
                  preferred_element_type=jnp.float32)
    bl0 = bl_ref[0, 0]
    bl1 = bl_ref[0, 1]
    s = ((b >> 1) + (b & 1)).astype(jnp.float32)
    r0 = bl0 + 0.5 * s * (bl1 - bl0)
    x = jnp.log(r0 / rij)
    out_t = a1f * jnp.exp(a2p * x)
    out_ref[...] = out_t[:out_ref.shape[0], :]


def _tc_node_body(z_ref, o_ref, out_ref):
    bn = z_ref.shape[0]
    z = z_ref[...].reshape(1, bn).astype(jnp.float32)
    c0 = o_ref[:, 0:1]
    c1 = o_ref[:, 1:2]
    nf_t = c0 + z * (c1 - c0)
    out_ref[...] = nf_t[:out_ref.shape[0], :]


def kernel(atomic_numbers, edge_index, edge_length, hopping_param,
           onsite_param, bond_length_list):
    n_nodes = atomic_numbers.shape[0]
    n_edges = edge_index.shape[1]
    edge_me = hopping_param.shape[1]
    node_me = onsite_param.shape[1]

    an = atomic_numbers.astype(jnp.int32)
    ei = edge_index.astype(jnp.int32)

    bond = _make_sc_bond(n_nodes, n_edges)(an, ei)

    a1t = jnp.zeros((16, 8), jnp.float32).at[:edge_me, :4].set(
        hopping_param[:, :, 0].T)
    a2t = jnp.zeros((16, 8), jnp.float32).at[:edge_me, :4].set(
        hopping_param[:, :, 1].T)
    ot = jnp.zeros((8, 2), jnp.float32).at[:node_me, :].set(
        onsite_param[:, :, 0].T)

    be = 65536
    grid_e = pl.cdiv(n_edges, be)
    ef_t = pl.pallas_call(
        _tc_edge_body,
        grid=(grid_e,),
        in_specs=[
            pl.BlockSpec((be,), lambda i: (i,)),
            pl.BlockSpec((be,), lambda i: (i,)),
            pl.BlockSpec((16, 8), lambda i: (0, 0)),
            pl.BlockSpec((16, 8), lambda i: (0, 0)),
            pl.BlockSpec((1, 2), lambda i: (0, 0)),
        ],
        out_specs=pl.BlockSpec((edge_me, be), lambda i: (0, i)),
        out_shape=jax.ShapeDtypeStruct((edge_me, n_edges), jnp.float32),
    )(edge_length, bond, a1t, a2t, bond_length_list.reshape(1, 2))

    nf_t = pl.pallas_call(
        _tc_node_body,
        in_specs=[
            pl.BlockSpec((n_nodes,), lambda: (0,)),
            pl.BlockSpec((8, 2), lambda: (0, 0)),
        ],
        out_specs=pl.BlockSpec((node_me, n_nodes), lambda: (0, 0)),
        out_shape=jax.ShapeDtypeStruct((node_me, n_nodes), jnp.float32),
    )(an, ot)

    return ef_t.T, nf_t.T


# speedup vs baseline: 497.1011x; 1.0477x over previous
import functools

import jax
import jax.numpy as jnp
from jax import lax
from jax.experimental import pallas as pl
from jax.experimental.pallas import tpu as pltpu
from jax.experimental.pallas import tpu_sc as plsc

RC = 5.0
W = 1.0
N_TYPES = 2

_NC = 2
_NS = 16
_L = 16


@functools.lru_cache(maxsize=None)
def _make_sc_bond(n_nodes: int, n_edges: int):
    nw = _NC * _NS
    ch = 6400
    assert n_edges % ch == 0
    n_chunks = n_edges // ch
    max_k = -(-n_chunks // nw)
    mesh = plsc.VectorSubcoreMesh(core_axis_name="c", subcore_axis_name="s")

    @functools.partial(
        pl.kernel,
        mesh=mesh,
        out_type=jax.ShapeDtypeStruct((n_edges,), jnp.int32),
        compiler_params=pltpu.CompilerParams(needs_layout_passes=False),
        scratch_types=[
            pltpu.VMEM((((n_nodes + 127) // 128) * 128,), jnp.int32),
            pltpu.VMEM((2, 2, ch), jnp.int32),
            pltpu.VMEM((2, ch), jnp.int32),
            pltpu.SemaphoreType.DMA,
            pltpu.SemaphoreType.DMA,
            pltpu.SemaphoreType.DMA,
            pltpu.SemaphoreType.DMA,
        ],
    )
    def sc_bond(an_hbm, ei_hbm, out_hbm, an_v, in_v, bo_v, si0, si1, so0, so1):
        wid = lax.axis_index("s") * _NC + lax.axis_index("c")
        sems_in = (si0, si1)
        sems_out = (so0, so1)
        pltpu.sync_copy(an_hbm, an_v.at[pl.ds(0, n_nodes)])

        def start_in(k):
            c = wid + nw * k

            @pl.when(c < n_chunks)
            def _():
                pltpu.async_copy(ei_hbm.at[:, pl.ds(c * ch, ch)],
                                 in_v.at[k % 2], sems_in[k % 2])

        start_in(0)
        for k in range(max_k):
            buf = k % 2
            c = wid + nw * k
            if k + 1 < max_k:
                start_in(k + 1)

            @pl.when(c < n_chunks)
            def _():
                pltpu.make_async_copy(ei_hbm.at[:, pl.ds(c * ch, ch)],
                                      in_v.at[buf], sems_in[buf]).wait()
                if k >= 2:
                    pltpu.make_async_copy(
                        bo_v.at[buf],
                        out_hbm.at[pl.ds((c - 2 * nw) * ch, ch)],
                        sems_out[buf]).wait()

                @plsc.parallel_loop(0, ch, step=_L, unroll=8)
                def body(i):
                    sl = pl.ds(i, _L)
                    zi = plsc.load_gather(an_v, [in_v[buf, 0, sl]])
                    zj = plsc.load_gather(an_v, [in_v[buf, 1, sl]])
                    bo_v[buf, sl] = zi * N_TYPES + zj
                pltpu.async_copy(bo_v.at[buf],
                                 out_hbm.at[pl.ds(c * ch, ch)], sems_out[buf])

        for k in range(max(0, max_k - 3), max_k):
            buf = k % 2
            c = wid + nw * k

            @pl.when((c < n_chunks) & (c + 2 * nw >= n_chunks))
            def _():
                pltpu.make_async_copy(bo_v.at[buf],
                                      out_hbm.at[pl.ds(c * ch, ch)],
                                      sems_out[buf]).wait()

    return sc_bond


def _tc_edge_body(rij_ref, bond_ref, a1_ref, a2_ref, bl_ref, out_ref):
    be = rij_ref.shape[0]
    rij = rij_ref[...].reshape(1, be)
    b = bond_ref[...].reshape(1, be)
    rows = lax.broadcasted_iota(jnp.int32, (8, be), 0)
    w_oh = (rows == b).astype(jnp.float32)
    fcut = 1.0 / (1.0 + jnp.exp((rij - RC + 5.0 * W) / W))
    a1f = jnp.dot(a1_ref[...], w_oh * fcut,
                  preferred_element_type=jnp.float32)
    a2p = jnp.dot(jnp.abs(a2_ref[...]) + 1.0, w_oh,
                  preferred_element_type=jnp.float32)
    bl0 = bl_ref[0, 0]
    bl1 = bl_ref[0, 1]
    s = ((b >> 1) + (b & 1)).astype(jnp.float32)
    r0 = bl0 + 0.5 * s * (bl1 - bl0)
    x = jnp.log(r0 / rij)
    out_t = a1f * jnp.exp(a2p * x)
    out_ref[...] = out_t[:out_ref.shape[0], :]


def _tc_node_body(z_ref, o_ref, out_ref):
    bn = z_ref.shape[0]
    z = z_ref[...].reshape(1, bn).astype(jnp.float32)
    c0 = o_ref[:, 0:1]
    c1 = o_ref[:, 1:2]
    nf_t = c0 + z * (c1 - c0)
    out_ref[...] = nf_t[:out_ref.shape[0], :]


def kernel(atomic_numbers, edge_index, edge_length, hopping_param,
           onsite_param, bond_length_list):
    n_nodes = atomic_numbers.shape[0]
    n_edges = edge_index.shape[1]
    edge_me = hopping_param.shape[1]
    node_me = onsite_param.shape[1]

    an = atomic_numbers.astype(jnp.int32)
    ei = edge_index.astype(jnp.int32)

    bond = _make_sc_bond(n_nodes, n_edges)(an, ei)

    a1t = jnp.zeros((16, 8), jnp.float32).at[:edge_me, :4].set(
        hopping_param[:, :, 0].T)
    a2t = jnp.zeros((16, 8), jnp.float32).at[:edge_me, :4].set(
        hopping_param[:, :, 1].T)
    ot = jnp.zeros((8, 2), jnp.float32).at[:node_me, :].set(
        onsite_param[:, :, 0].T)

    be = 131072
    grid_e = pl.cdiv(n_edges, be)
    ef_t = pl.pallas_call(
        _tc_edge_body,
        grid=(grid_e,),
        in_specs=[
            pl.BlockSpec((be,), lambda i: (i,)),
            pl.BlockSpec((be,), lambda i: (i,)),
            pl.BlockSpec((16, 8), lambda i: (0, 0)),
            pl.BlockSpec((16, 8), lambda i: (0, 0)),
            pl.BlockSpec((1, 2), lambda i: (0, 0)),
        ],
        out_specs=pl.BlockSpec((edge_me, be), lambda i: (0, i)),
        out_shape=jax.ShapeDtypeStruct((edge_me, n_edges), jnp.float32),
    )(edge_length, bond, a1t, a2t, bond_length_list.reshape(1, 2))

    nf_t = pl.pallas_call(
        _tc_node_body,
        in_specs=[
            pl.BlockSpec((n_nodes,), lambda: (0,)),
            pl.BlockSpec((8, 2), lambda: (0, 0)),
        ],
        out_specs=pl.BlockSpec((node_me, n_nodes), lambda: (0, 0)),
        out_shape=jax.ShapeDtypeStruct((node_me, n_nodes), jnp.float32),
    )(an, ot)

    return ef_t.T, nf_t.T
